# Initial kernel scaffold; baseline (speedup 1.0000x reference)
import jax
import jax.numpy as jnp
from jax.experimental import pallas as pl


def kernel(vad, labels, W_blocks):
    raise NotImplementedError("write your pallas kernel here")


# speedup vs baseline: 2.6752x; 2.6752x over previous
import functools

import jax
import jax.numpy as jnp
from jax import lax
from jax.experimental import pallas as pl
from jax.experimental.pallas import tpu as pltpu
from jax.experimental.pallas import tpu_sc as plsc

B = 16384
NUM_CLASSES = 1024
IN_DIM = 3
OUT_DIM = 768
LANES = 16
NUM_CORES = 2
NUM_SUBCORES = 16
NW = NUM_CORES * NUM_SUBCORES
BPW = B // NW
K = 32
NCHUNK = BPW // K
NT = OUT_DIM // LANES


def _body(wt_hbm, vad_hbm, labels_hbm, out_hbm, idx_v, vad_v, rows_v, out_v,
          sem):
    wid = lax.axis_index("s") * NUM_CORES + lax.axis_index("c")
    base = wid * BPW

    def sample_body(s, carry):
        p = jnp.full((LANES,), IN_DIM * s, jnp.int32)
        v0 = plsc.load_gather(vad_v, [p])
        v1 = plsc.load_gather(vad_v, [p + 1])
        v2 = plsc.load_gather(vad_v, [p + 2])
        for t in range(NT):
            sl = pl.ds(t * LANES, LANES)
            w0 = rows_v[s, pl.ds(t * LANES, LANES)]
            w1 = rows_v[s, pl.ds(OUT_DIM + t * LANES, LANES)]
            w2 = rows_v[s, pl.ds(2 * OUT_DIM + t * LANES, LANES)]
            out_v[s, sl] = w0 * v0 + w1 * v1 + w2 * v2
        return carry

    def chunk_body(c, carry):
        cbase = pl.multiple_of(base + c * K, K)
        pltpu.sync_copy(labels_hbm.at[pl.ds(cbase, K)], idx_v)
        pltpu.sync_copy(vad_hbm.at[pl.ds(cbase * IN_DIM, K * IN_DIM)], vad_v)
        pltpu.async_copy(wt_hbm.at[idx_v], rows_v, sem).wait()
        lax.fori_loop(0, K, sample_body, 0)
        pltpu.sync_copy(out_v, out_hbm.at[pl.ds(cbase, K)])
        return carry

    lax.fori_loop(0, NCHUNK, chunk_body, 0)


@jax.jit
def _run(wt, vad, labels):
    mesh = plsc.VectorSubcoreMesh(core_axis_name="c", subcore_axis_name="s")
    kfn = pl.kernel(
        _body,
        out_type=jax.ShapeDtypeStruct((B, OUT_DIM), jnp.float32),
        mesh=mesh,
        compiler_params=pltpu.CompilerParams(needs_layout_passes=False),
        scratch_types=[
            pltpu.VMEM((K,), jnp.int32),
            pltpu.VMEM((K * IN_DIM,), jnp.float32),
            pltpu.VMEM((K, IN_DIM * OUT_DIM), jnp.float32),
            pltpu.VMEM((K, OUT_DIM), jnp.float32),
            pltpu.SemaphoreType.DMA,
        ],
    )
    return kfn(wt, vad, labels)


def kernel(vad, labels, W_blocks):
    wt = jnp.transpose(W_blocks, (0, 2, 1)).reshape(NUM_CLASSES, -1)
    return _run(wt, vad.reshape(-1), labels.astype(jnp.int32))


# speedup vs baseline: 4.7099x; 1.7606x over previous
import functools

import jax
import jax.numpy as jnp
from jax import lax
from jax.experimental import pallas as pl
from jax.experimental.pallas import tpu as pltpu
from jax.experimental.pallas import tpu_sc as plsc

B = 16384
NUM_CLASSES = 1024
IN_DIM = 3
OUT_DIM = 768
LANES = 16
NUM_CORES = 2
NUM_SUBCORES = 16
NW = NUM_CORES * NUM_SUBCORES
BPW = B // NW
K = 32
NCHUNK = BPW // K
NT = OUT_DIM // LANES


def _body(wt_hbm, vad_hbm, labels_hbm, out_hbm, idx_v, vad_v, rows_v, out_v,
          sem):
    wid = lax.axis_index("s") * NUM_CORES + lax.axis_index("c")
    base = wid * BPW

    GT = 4

    def sample_body(s):
        p = jnp.full((LANES,), IN_DIM * s, jnp.int32)
        v0 = plsc.load_gather(vad_v, [p])
        v1 = plsc.load_gather(vad_v, [p + 1])
        v2 = plsc.load_gather(vad_v, [p + 2])
        for g in range(NT // GT):
            ws = []
            for j in range(GT):
                t = g * GT + j
                ws.append((
                    rows_v[s, pl.ds(t * LANES, LANES)],
                    rows_v[s, pl.ds(OUT_DIM + t * LANES, LANES)],
                    rows_v[s, pl.ds(2 * OUT_DIM + t * LANES, LANES)],
                ))
            for j in range(GT):
                t = g * GT + j
                w0, w1, w2 = ws[j]
                out_v[s, pl.ds(t * LANES, LANES)] = (
                    w0 * v0 + w1 * v1 + w2 * v2)

    def chunk_body(c, carry):
        cbase = pl.multiple_of(base + c * K, K)
        pltpu.sync_copy(labels_hbm.at[pl.ds(cbase, K)], idx_v)
        pltpu.sync_copy(vad_hbm.at[pl.ds(cbase * IN_DIM, K * IN_DIM)], vad_v)
        pltpu.async_copy(wt_hbm.at[idx_v], rows_v, sem).wait()
        plsc.parallel_loop(0, K, step=1, unroll=2)(sample_body)
        pltpu.sync_copy(out_v, out_hbm.at[pl.ds(cbase, K)])
        return carry

    lax.fori_loop(0, NCHUNK, chunk_body, 0)


@jax.jit
def _run(wt, vad, labels):
    mesh = plsc.VectorSubcoreMesh(core_axis_name="c", subcore_axis_name="s")
    kfn = pl.kernel(
        _body,
        out_type=jax.ShapeDtypeStruct((B, OUT_DIM), jnp.float32),
        mesh=mesh,
        compiler_params=pltpu.CompilerParams(needs_layout_passes=False),
        scratch_types=[
            pltpu.VMEM((K,), jnp.int32),
            pltpu.VMEM((K * IN_DIM,), jnp.float32),
            pltpu.VMEM((K, IN_DIM * OUT_DIM), jnp.float32),
            pltpu.VMEM((K, OUT_DIM), jnp.float32),
            pltpu.SemaphoreType.DMA,
        ],
    )
    return kfn(wt, vad, labels)


def kernel(vad, labels, W_blocks):
    wt = jnp.transpose(W_blocks, (0, 2, 1)).reshape(NUM_CLASSES, -1)
    return _run(wt, vad.reshape(-1), labels.astype(jnp.int32))


# speedup vs baseline: 5.5498x; 1.1783x over previous
import functools

import jax
import jax.numpy as jnp
from jax import lax
from jax.experimental import pallas as pl
from jax.experimental.pallas import tpu as pltpu
from jax.experimental.pallas import tpu_sc as plsc

B = 16384
NUM_CLASSES = 1024
IN_DIM = 3
OUT_DIM = 768
LANES = 16
NUM_CORES = 2
NUM_SUBCORES = 16
NW = NUM_CORES * NUM_SUBCORES
BPW = B // NW
K = 16
NCHUNK = BPW // K
NPAIR = NCHUNK // 2
NT = OUT_DIM // LANES
ROW = IN_DIM * OUT_DIM


def _body(wt_hbm, vad_hbm, labels_hbm, out_hbm, idx_v, vad_v, rows_a, rows_b,
          out_v, sem_a, sem_b):
    wid = lax.axis_index("s") * NUM_CORES + lax.axis_index("c")
    base = wid * BPW

    pltpu.sync_copy(labels_hbm.at[pl.ds(base, BPW)], idx_v)
    pltpu.sync_copy(vad_hbm.at[pl.ds(base * IN_DIM, BPW * IN_DIM)], vad_v)

    GT = 4

    def sample_body(rows, cbase, s):
        p = jnp.full((LANES,), IN_DIM * (cbase + s), jnp.int32)
        v0 = plsc.load_gather(vad_v, [p])
        v1 = plsc.load_gather(vad_v, [p + 1])
        v2 = plsc.load_gather(vad_v, [p + 2])
        for g in range(NT // GT):
            ws = []
            for j in range(GT):
                t = g * GT + j
                ws.append((
                    rows[s, pl.ds(t * LANES, LANES)],
                    rows[s, pl.ds(OUT_DIM + t * LANES, LANES)],
                    rows[s, pl.ds(2 * OUT_DIM + t * LANES, LANES)],
                ))
            for j in range(GT):
                t = g * GT + j
                w0, w1, w2 = ws[j]
                out_v[s, pl.ds(t * LANES, LANES)] = (
                    w0 * v0 + w1 * v1 + w2 * v2)

    def issue_gather(c, rows, sem):
        return pltpu.async_copy(
            wt_hbm.at[idx_v.at[pl.ds(c * K, K)]], rows, sem)

    def half(c, c_next, rows, sem, rows_next, sem_next):
        pltpu.make_async_copy(
            wt_hbm.at[idx_v.at[pl.ds(c * K, K)]], rows, sem).wait()
        gnext = issue_gather(c_next, rows_next, sem_next)
        cbase = pl.multiple_of(c * K, K)
        plsc.parallel_loop(0, K, step=1, unroll=2)(
            functools.partial(sample_body, rows, cbase))
        pltpu.sync_copy(out_v, out_hbm.at[pl.ds(base + c * K, K)])
        return gnext

    issue_gather(0, rows_a, sem_a)

    def pair_body(j, carry):
        c0 = j * 2
        half(c0, c0 + 1, rows_a, sem_a, rows_b, sem_b)
        half(c0 + 1, (c0 + 2) % NCHUNK, rows_b, sem_b, rows_a, sem_a)
        return carry

    lax.fori_loop(0, NPAIR, pair_body, 0)
    pltpu.make_async_copy(
        wt_hbm.at[idx_v.at[pl.ds(0, K)]], rows_a, sem_a).wait()


@jax.jit
def _run(wt, vad, labels):
    mesh = plsc.VectorSubcoreMesh(core_axis_name="c", subcore_axis_name="s")
    kfn = pl.kernel(
        _body,
        out_type=jax.ShapeDtypeStruct((B, OUT_DIM), jnp.float32),
        mesh=mesh,
        compiler_params=pltpu.CompilerParams(needs_layout_passes=False),
        scratch_types=[
            pltpu.VMEM((BPW,), jnp.int32),
            pltpu.VMEM((BPW * IN_DIM,), jnp.float32),
            pltpu.VMEM((K, ROW), jnp.float32),
            pltpu.VMEM((K, ROW), jnp.float32),
            pltpu.VMEM((K, OUT_DIM), jnp.float32),
            pltpu.SemaphoreType.DMA,
            pltpu.SemaphoreType.DMA,
        ],
    )
    return kfn(wt, vad, labels)


def kernel(vad, labels, W_blocks):
    wt = jnp.transpose(W_blocks, (0, 2, 1)).reshape(NUM_CLASSES, -1)
    return _run(wt, vad.reshape(-1), labels.astype(jnp.int32))


# speedup vs baseline: 5.7033x; 1.0277x over previous
import functools

import jax
import jax.numpy as jnp
from jax import lax
from jax.experimental import pallas as pl
from jax.experimental.pallas import tpu as pltpu
from jax.experimental.pallas import tpu_sc as plsc

B = 16384
NUM_CLASSES = 1024
IN_DIM = 3
OUT_DIM = 768
LANES = 16
NUM_CORES = 2
NUM_SUBCORES = 16
NW = NUM_CORES * NUM_SUBCORES
BPW = B // NW
K = 16
NCHUNK = BPW // K
NPAIR = NCHUNK // 2
NT = OUT_DIM // LANES
ROW = IN_DIM * OUT_DIM


def _body(wt_hbm, vad_hbm, labels_hbm, out_hbm, idx_v, vad_v, rows_a, rows_b,
          out_a, out_b, sem_a, sem_b, sem_oa, sem_ob):
    wid = lax.axis_index("s") * NUM_CORES + lax.axis_index("c")
    base = wid * BPW

    pltpu.sync_copy(labels_hbm.at[pl.ds(base, BPW)], idx_v)
    pltpu.sync_copy(vad_hbm.at[pl.ds(base * IN_DIM, BPW * IN_DIM)], vad_v)

    GT = 4

    def sample_body(rows, out_v, cbase, s):
        p = jnp.full((LANES,), IN_DIM * (cbase + s), jnp.int32)
        v0 = plsc.load_gather(vad_v, [p])
        v1 = plsc.load_gather(vad_v, [p + 1])
        v2 = plsc.load_gather(vad_v, [p + 2])
        for g in range(NT // GT):
            ws = []
            for j in range(GT):
                t = g * GT + j
                ws.append((
                    rows[s, pl.ds(t * LANES, LANES)],
                    rows[s, pl.ds(OUT_DIM + t * LANES, LANES)],
                    rows[s, pl.ds(2 * OUT_DIM + t * LANES, LANES)],
                ))
            for j in range(GT):
                t = g * GT + j
                w0, w1, w2 = ws[j]
                out_v[s, pl.ds(t * LANES, LANES)] = (
                    w0 * v0 + w1 * v1 + w2 * v2)

    def issue_gather(c, rows, sem):
        return pltpu.async_copy(
            wt_hbm.at[idx_v.at[pl.ds(c * K, K)]], rows, sem)

    def out_copy(c, out_v, sem_o):
        return pltpu.make_async_copy(
            out_v, out_hbm.at[pl.ds(base + c * K, K)], sem_o)

    def half(c, c_next, rows, sem, rows_next, sem_next, out_v, sem_o):
        pltpu.make_async_copy(
            wt_hbm.at[idx_v.at[pl.ds(c * K, K)]], rows, sem).wait()
        issue_gather(c_next, rows_next, sem_next)
        cbase = pl.multiple_of(c * K, K)
        plsc.parallel_loop(0, K, step=1, unroll=2)(
            functools.partial(sample_body, rows, out_v, cbase))
        out_copy(c, out_v, sem_o).start()

    issue_gather(0, rows_a, sem_a)

    def pair_body(j, carry):
        c0 = j * 2

        @pl.when(j > 0)
        def _drain_prev():
            out_copy(c0 - 2, out_a, sem_oa).wait()
            out_copy(c0 - 1, out_b, sem_ob).wait()

        half(c0, c0 + 1, rows_a, sem_a, rows_b, sem_b, out_a, sem_oa)
        half(c0 + 1, (c0 + 2) % NCHUNK, rows_b, sem_b, rows_a, sem_a,
             out_b, sem_ob)
        return carry

    lax.fori_loop(0, NPAIR, pair_body, 0)
    pltpu.make_async_copy(
        wt_hbm.at[idx_v.at[pl.ds(0, K)]], rows_a, sem_a).wait()
    out_copy(NCHUNK - 2, out_a, sem_oa).wait()
    out_copy(NCHUNK - 1, out_b, sem_ob).wait()


@jax.jit
def _run(wt, vad, labels):
    mesh = plsc.VectorSubcoreMesh(core_axis_name="c", subcore_axis_name="s")
    kfn = pl.kernel(
        _body,
        out_type=jax.ShapeDtypeStruct((B, OUT_DIM), jnp.float32),
        mesh=mesh,
        compiler_params=pltpu.CompilerParams(needs_layout_passes=False),
        scratch_types=[
            pltpu.VMEM((BPW,), jnp.int32),
            pltpu.VMEM((BPW * IN_DIM,), jnp.float32),
            pltpu.VMEM((K, ROW), jnp.float32),
            pltpu.VMEM((K, ROW), jnp.float32),
            pltpu.VMEM((K, OUT_DIM), jnp.float32),
            pltpu.VMEM((K, OUT_DIM), jnp.float32),
            pltpu.SemaphoreType.DMA,
            pltpu.SemaphoreType.DMA,
            pltpu.SemaphoreType.DMA,
            pltpu.SemaphoreType.DMA,
        ],
    )
    return kfn(wt, vad, labels)


def kernel(vad, labels, W_blocks):
    wt = jnp.transpose(W_blocks, (0, 2, 1)).reshape(NUM_CLASSES, -1)
    return _run(wt, vad.reshape(-1), labels.astype(jnp.int32))
